# Initial kernel scaffold; baseline (speedup 1.0000x reference)
import functools

import jax
import jax.numpy as jnp
from jax import lax
from jax.experimental import pallas as pl
from jax.experimental.pallas import tpu as pltpu
from jax.experimental.pallas import tpu_sc as plsc

T = 2048
D = 1024
E = 8
CAP = 640
S = E * CAP
DFF = 3072
DH = DFF // 2
NC, NS = 2, 16
NW = NC * NS
RPT = S // NW
GCH = 80
TPT = T // NW
CCH = 32

_MESH = plsc.VectorSubcoreMesh(core_axis_name="c", subcore_axis_name="s")


def _route_body(x_ref, wg_ref, slot0_ref, slot1_ref, keep0_ref, keep1_ref,
                w0r_ref, w1r_ref):
    x = x_ref[...]
    wg = wg_ref[...]
    logits = jnp.dot(x, wg, preferred_element_type=jnp.float32)
    lt = logits.T
    row = lax.broadcasted_iota(jnp.int32, (E, T), 0)
    m0 = jnp.max(lt, axis=0, keepdims=True)
    i0 = jnp.min(jnp.where(lt == m0, row, E), axis=0, keepdims=True)
    sel0 = row == i0
    masked = jnp.where(sel0, -jnp.inf, lt)
    m1 = jnp.max(masked, axis=0, keepdims=True)
    i1 = jnp.min(jnp.where(masked == m1, row, E), axis=0, keepdims=True)
    sel1 = row == i1
    ez = jnp.exp(lt - m0)
    ssum = jnp.sum(ez, axis=0, keepdims=True)
    w0 = 1.0 / ssum
    w1 = jnp.exp(m1 - m0) / ssum
    cnt = sel0.astype(jnp.float32) + sel1.astype(jnp.float32)
    c = cnt
    sh = 1
    while sh < T:
        c = c + jnp.concatenate(
            [jnp.zeros((E, sh), jnp.float32), c[:, :T - sh]], axis=1)
        sh *= 2
    cex = c - cnt
    pos0 = jnp.sum(jnp.where(sel0, cex, 0.0), axis=0, keepdims=True)
    pos1 = jnp.sum(jnp.where(sel1, cex, 0.0), axis=0, keepdims=True)
    keep0 = (pos0 < CAP).astype(jnp.int32)
    keep1 = (pos1 < CAP).astype(jnp.int32)
    p0 = jnp.minimum(pos0, CAP - 1.0).astype(jnp.int32)
    p1 = jnp.minimum(pos1, CAP - 1.0).astype(jnp.int32)
    slot0_ref[...] = i0 * CAP + p0
    slot1_ref[...] = i1 * CAP + p1
    keep0_ref[...] = keep0
    keep1_ref[...] = keep1
    cw0 = (w0 * keep0.astype(jnp.float32)).reshape(T, 1)
    cw1 = (w1 * keep1.astype(jnp.float32)).reshape(T, 1)
    w0r_ref[...] = jnp.broadcast_to(cw0, (T, 16))
    w1r_ref[...] = jnp.broadcast_to(cw1, (T, 16))


def _route(x2, w_gate):
    return pl.pallas_call(
        _route_body,
        out_shape=[
            jax.ShapeDtypeStruct((1, T), jnp.int32),
            jax.ShapeDtypeStruct((1, T), jnp.int32),
            jax.ShapeDtypeStruct((1, T), jnp.int32),
            jax.ShapeDtypeStruct((1, T), jnp.int32),
            jax.ShapeDtypeStruct((T, 16), jnp.float32),
            jax.ShapeDtypeStruct((T, 16), jnp.float32),
        ],
    )(x2, w_gate)


@functools.partial(
    pl.kernel,
    out_type=jax.ShapeDtypeStruct((S,), jnp.int32),
    mesh=_MESH,
    scratch_types=[
        pltpu.VMEM((T,), jnp.int32),
        pltpu.VMEM((T,), jnp.int32),
        pltpu.VMEM((T,), jnp.int32),
        pltpu.VMEM((T,), jnp.int32),
        pltpu.VMEM((S,), jnp.int32),
    ],
)
def _scatter(slot0_hbm, slot1_hbm, keep0_hbm, keep1_hbm, out_hbm,
             s0_v, s1_v, k0_v, k1_v, st_v):
    wid = lax.axis_index("s") * NC + lax.axis_index("c")

    @pl.when(wid == 0)
    def _():
        pltpu.sync_copy(slot0_hbm.at[0], s0_v)
        pltpu.sync_copy(slot1_hbm.at[0], s1_v)
        pltpu.sync_copy(keep0_hbm.at[0], k0_v)
        pltpu.sync_copy(keep1_hbm.at[0], k1_v)
        zero = jnp.zeros((16,), jnp.int32)

        def zbody(i, carry):
            st_v[pl.ds(i * 16, 16)] = zero
            return carry

        lax.fori_loop(0, S // 16, zbody, 0)
        lanes = lax.iota(jnp.int32, 16)

        def body(i, carry):
            base = i * 16
            toks = base + lanes
            plsc.store_scatter(st_v, [s0_v[pl.ds(base, 16)]], toks,
                               mask=k0_v[pl.ds(base, 16)] != 0)
            plsc.store_scatter(st_v, [s1_v[pl.ds(base, 16)]], toks,
                               mask=k1_v[pl.ds(base, 16)] != 0)
            return carry

        lax.fori_loop(0, T // 16, body, 0)
        pltpu.sync_copy(st_v, out_hbm)


@functools.partial(
    pl.kernel,
    out_type=jax.ShapeDtypeStruct((S, D), jnp.float32),
    mesh=_MESH,
    scratch_types=[
        pltpu.VMEM((RPT,), jnp.int32),
        pltpu.VMEM((GCH, D), jnp.float32),
        pltpu.SemaphoreType.DMA,
    ],
)
def _gather(st_hbm, x_hbm, out_hbm, idx_v, rows_v, sem):
    wid = lax.axis_index("s") * NC + lax.axis_index("c")
    base = wid * RPT
    pltpu.sync_copy(st_hbm.at[pl.ds(base, RPT)], idx_v)
    for ch in range(RPT // GCH):
        pltpu.async_copy(x_hbm.at[idx_v.at[pl.ds(ch * GCH, GCH)]],
                         rows_v, sem).wait()
        pltpu.sync_copy(rows_v, out_hbm.at[pl.ds(base + ch * GCH, GCH)])


def _mlp_body(xb_ref, fc_ref, pj_ref, out_ref):
    a = xb_ref[...].astype(jnp.bfloat16)
    w1 = fc_ref[0].astype(jnp.bfloat16)
    h = jnp.dot(a, w1, preferred_element_type=jnp.float32)
    u = h[:, :DH]
    g = h[:, DH:]
    hh = (u * lax.logistic(u) * g).astype(jnp.bfloat16)
    w2 = pj_ref[0].astype(jnp.bfloat16)
    out_ref[...] = jnp.dot(hh, w2, preferred_element_type=jnp.float32)


def _mlp(exp_x, c_fc, c_proj):
    return pl.pallas_call(
        _mlp_body,
        grid=(E,),
        in_specs=[
            pl.BlockSpec((CAP, D), lambda e: (e, 0)),
            pl.BlockSpec((1, D, DFF), lambda e: (e, 0, 0)),
            pl.BlockSpec((1, DH, D), lambda e: (e, 0, 0)),
        ],
        out_specs=pl.BlockSpec((CAP, D), lambda e: (e, 0)),
        out_shape=jax.ShapeDtypeStruct((S, D), jnp.float32),
    )(exp_x, c_fc, c_proj)


@functools.partial(
    pl.kernel,
    out_type=jax.ShapeDtypeStruct((T, D), jnp.float32),
    mesh=_MESH,
    scratch_types=[
        pltpu.VMEM((CCH,), jnp.int32),
        pltpu.VMEM((CCH,), jnp.int32),
        pltpu.VMEM((CCH, 16), jnp.float32),
        pltpu.VMEM((CCH, 16), jnp.float32),
        pltpu.VMEM((CCH, D), jnp.float32),
        pltpu.VMEM((CCH, D), jnp.float32),
        pltpu.VMEM((CCH, D), jnp.float32),
        pltpu.SemaphoreType.DMA,
    ],
)
def _combine(slot0_hbm, slot1_hbm, w0r_hbm, w1r_hbm, eo_hbm, out_hbm,
             i0_v, i1_v, w0_v, w1_v, r0_v, r1_v, o_v, sem):
    wid = lax.axis_index("s") * NC + lax.axis_index("c")
    for half in range(TPT // CCH):
        tb = wid * TPT + half * CCH
        pltpu.sync_copy(slot0_hbm.at[0, pl.ds(tb, CCH)], i0_v)
        pltpu.sync_copy(slot1_hbm.at[0, pl.ds(tb, CCH)], i1_v)
        pltpu.sync_copy(w0r_hbm.at[pl.ds(tb, CCH)], w0_v)
        pltpu.sync_copy(w1r_hbm.at[pl.ds(tb, CCH)], w1_v)
        pltpu.async_copy(eo_hbm.at[i0_v], r0_v, sem).wait()
        pltpu.async_copy(eo_hbm.at[i1_v], r1_v, sem).wait()

        def tok_body(t, carry):
            wv0 = w0_v[t, :]
            wv1 = w1_v[t, :]
            for cidx in range(D // 16):
                a = r0_v[t, pl.ds(cidx * 16, 16)]
                b = r1_v[t, pl.ds(cidx * 16, 16)]
                o_v[t, pl.ds(cidx * 16, 16)] = a * wv0 + b * wv1
            return carry

        lax.fori_loop(0, CCH, tok_body, 0)
        pltpu.sync_copy(o_v, out_hbm.at[pl.ds(tb, CCH)])


def kernel(x, w_gate, c_fc, c_proj):
    x2 = x.reshape(T, D)
    slot0, slot1, keep0, keep1, w0r, w1r = _route(x2, w_gate)
    slot_tok = _scatter(slot0, slot1, keep0, keep1)
    exp_x = _gather(slot_tok, x2)
    exp_out = _mlp(exp_x, c_fc, c_proj)
    out = _combine(slot0, slot1, w0r, w1r, exp_out)
    return out.reshape(1, T, D)


# speedup vs baseline: 1.2772x; 1.2772x over previous
import functools

import jax
import jax.numpy as jnp
from jax import lax
from jax.experimental import pallas as pl
from jax.experimental.pallas import tpu as pltpu
from jax.experimental.pallas import tpu_sc as plsc

T = 2048
D = 1024
E = 8
CAP = 640
S = E * CAP
DFF = 3072
DH = DFF // 2
NC, NS = 2, 16
NW = NC * NS
RPT = S // NW
GCH = 80
TPT = T // NW
CCH = 32


def _route_body(x_ref, wg_ref, slot0_ref, slot1_ref, keep0_ref, keep1_ref,
                w0r_ref, w1r_ref):
    x = x_ref[...]
    wg = wg_ref[...]
    logits = jnp.dot(x, wg, preferred_element_type=jnp.float32)
    lt = logits.T
    row = lax.broadcasted_iota(jnp.int32, (E, T), 0)
    m0 = jnp.max(lt, axis=0, keepdims=True)
    i0 = jnp.min(jnp.where(lt == m0, row, E), axis=0, keepdims=True)
    sel0 = row == i0
    masked = jnp.where(sel0, -jnp.inf, lt)
    m1 = jnp.max(masked, axis=0, keepdims=True)
    i1 = jnp.min(jnp.where(masked == m1, row, E), axis=0, keepdims=True)
    sel1 = row == i1
    ez = jnp.exp(lt - m0)
    ssum = jnp.sum(ez, axis=0, keepdims=True)
    w0 = 1.0 / ssum
    w1 = jnp.exp(m1 - m0) / ssum
    cnt = sel0.astype(jnp.float32) + sel1.astype(jnp.float32)
    c = cnt
    sh = 1
    while sh < T:
        c = c + jnp.concatenate(
            [jnp.zeros((E, sh), jnp.float32), c[:, :T - sh]], axis=1)
        sh *= 2
    cex = c - cnt
    pos0 = jnp.sum(jnp.where(sel0, cex, 0.0), axis=0, keepdims=True)
    pos1 = jnp.sum(jnp.where(sel1, cex, 0.0), axis=0, keepdims=True)
    keep0 = (pos0 < CAP).astype(jnp.int32)
    keep1 = (pos1 < CAP).astype(jnp.int32)
    p0 = jnp.minimum(pos0, CAP - 1.0).astype(jnp.int32)
    p1 = jnp.minimum(pos1, CAP - 1.0).astype(jnp.int32)
    slot0_ref[...] = i0 * CAP + p0
    slot1_ref[...] = i1 * CAP + p1
    keep0_ref[...] = keep0
    keep1_ref[...] = keep1
    cw0 = (w0 * keep0.astype(jnp.float32)).reshape(T, 1)
    cw1 = (w1 * keep1.astype(jnp.float32)).reshape(T, 1)
    w0r_ref[...] = jnp.broadcast_to(cw0, (T, 16))
    w1r_ref[...] = jnp.broadcast_to(cw1, (T, 16))


def _route(x2, w_gate):
    return pl.pallas_call(
        _route_body,
        out_shape=[
            jax.ShapeDtypeStruct((1, T), jnp.int32),
            jax.ShapeDtypeStruct((1, T), jnp.int32),
            jax.ShapeDtypeStruct((1, T), jnp.int32),
            jax.ShapeDtypeStruct((1, T), jnp.int32),
            jax.ShapeDtypeStruct((T, 16), jnp.float32),
            jax.ShapeDtypeStruct((T, 16), jnp.float32),
        ],
    )(x2, w_gate)


def _scatter_body(slot0_hbm, slot1_hbm, keep0_hbm, keep1_hbm, out_hbm,
                  s_v, k_v, idx_v, tok_v):
    wid = lax.axis_index("s") * NC + lax.axis_index("c")
    epw = (2 * T) // NW
    tok_base = (wid % (T // epw)) * epw
    lanes = lax.iota(jnp.int32, 16)

    @pl.when(wid < T // epw)
    def _():
        pltpu.sync_copy(slot0_hbm.at[0, pl.ds(tok_base, epw)], s_v)
        pltpu.sync_copy(keep0_hbm.at[0, pl.ds(tok_base, epw)], k_v)

    @pl.when(wid >= T // epw)
    def _():
        pltpu.sync_copy(slot1_hbm.at[0, pl.ds(tok_base, epw)], s_v)
        pltpu.sync_copy(keep1_hbm.at[0, pl.ds(tok_base, epw)], k_v)

    for ch in range(epw // 16):
        sl = pl.ds(ch * 16, 16)
        idx_v[sl] = jnp.where(k_v[sl] != 0, s_v[sl], S)
        tok_v[sl] = tok_base + ch * 16 + lanes
    pltpu.sync_copy(tok_v, out_hbm.at[idx_v])


def _gather_body(st_hbm, x_hbm, out_hbm, idx_v, rows_v, sem):
    wid = lax.axis_index("s") * NC + lax.axis_index("c")
    base = wid * RPT
    pltpu.sync_copy(st_hbm.at[pl.ds(base, RPT)], idx_v)
    for ch in range(RPT // 16):
        sl = pl.ds(ch * 16, 16)
        idx_v[sl] = jnp.clip(idx_v[sl], 0, T - 1)
    for ch in range(RPT // GCH):
        pltpu.async_copy(x_hbm.at[idx_v.at[pl.ds(ch * GCH, GCH)]],
                         rows_v, sem).wait()
        pltpu.sync_copy(rows_v, out_hbm.at[pl.ds(base + ch * GCH, GCH)])


def _mlp_body(xb_ref, fc_ref, pj_ref, out_ref):
    a = xb_ref[...].astype(jnp.bfloat16)
    w1 = fc_ref[0].astype(jnp.bfloat16)
    h = jnp.dot(a, w1, preferred_element_type=jnp.float32)
    u = h[:, :DH]
    g = h[:, DH:]
    hh = (u * lax.logistic(u) * g).astype(jnp.bfloat16)
    w2 = pj_ref[0].astype(jnp.bfloat16)
    out_ref[...] = jnp.dot(hh, w2, preferred_element_type=jnp.float32)


def _mlp(exp_x, c_fc, c_proj):
    return pl.pallas_call(
        _mlp_body,
        grid=(E,),
        in_specs=[
            pl.BlockSpec((CAP, D), lambda e: (e, 0)),
            pl.BlockSpec((1, D, DFF), lambda e: (e, 0, 0)),
            pl.BlockSpec((1, DH, D), lambda e: (e, 0, 0)),
        ],
        out_specs=pl.BlockSpec((CAP, D), lambda e: (e, 0)),
        out_shape=jax.ShapeDtypeStruct((S, D), jnp.float32),
    )(exp_x, c_fc, c_proj)


def _combine_body(slot0_hbm, slot1_hbm, w0r_hbm, w1r_hbm, eo_hbm, out_hbm,
             i0_v, i1_v, w0_v, w1_v, r0_v, r1_v, o_v, sem):
    wid = lax.axis_index("s") * NC + lax.axis_index("c")
    for half in range(TPT // CCH):
        tb = wid * TPT + half * CCH
        pltpu.sync_copy(slot0_hbm.at[0, pl.ds(tb, CCH)], i0_v)
        pltpu.sync_copy(slot1_hbm.at[0, pl.ds(tb, CCH)], i1_v)
        pltpu.sync_copy(w0r_hbm.at[pl.ds(tb, CCH)], w0_v)
        pltpu.sync_copy(w1r_hbm.at[pl.ds(tb, CCH)], w1_v)
        pltpu.async_copy(eo_hbm.at[i0_v], r0_v, sem).wait()
        pltpu.async_copy(eo_hbm.at[i1_v], r1_v, sem).wait()

        def tok_body(t, carry):
            wv0 = w0_v[t, :]
            wv1 = w1_v[t, :]
            for cidx in range(D // 16):
                a = r0_v[t, pl.ds(cidx * 16, 16)]
                b = r1_v[t, pl.ds(cidx * 16, 16)]
                o_v[t, pl.ds(cidx * 16, 16)] = a * wv0 + b * wv1
            return carry

        lax.fori_loop(0, CCH, tok_body, 0)
        pltpu.sync_copy(o_v, out_hbm.at[pl.ds(tb, CCH)])


@functools.cache
def _sc_kernels():
    mesh = plsc.VectorSubcoreMesh(core_axis_name="c", subcore_axis_name="s",
                                  num_cores=NC, num_subcores=NS)
    scatter = pl.kernel(
        _scatter_body,
        out_type=jax.ShapeDtypeStruct((S + 8,), jnp.int32),
        mesh=mesh,
        scratch_types=[
            pltpu.VMEM(((2 * T) // NW,), jnp.int32),
            pltpu.VMEM(((2 * T) // NW,), jnp.int32),
            pltpu.VMEM(((2 * T) // NW,), jnp.int32),
            pltpu.VMEM(((2 * T) // NW,), jnp.int32),
        ],
    )
    gather = pl.kernel(
        _gather_body,
        out_type=jax.ShapeDtypeStruct((S, D), jnp.float32),
        mesh=mesh,
        scratch_types=[
            pltpu.VMEM((RPT,), jnp.int32),
            pltpu.VMEM((GCH, D), jnp.float32),
            pltpu.SemaphoreType.DMA,
        ],
    )
    combine = pl.kernel(
        _combine_body,
        out_type=jax.ShapeDtypeStruct((T, D), jnp.float32),
        mesh=mesh,
        scratch_types=[
            pltpu.VMEM((CCH,), jnp.int32),
            pltpu.VMEM((CCH,), jnp.int32),
            pltpu.VMEM((CCH, 16), jnp.float32),
            pltpu.VMEM((CCH, 16), jnp.float32),
            pltpu.VMEM((CCH, D), jnp.float32),
            pltpu.VMEM((CCH, D), jnp.float32),
            pltpu.VMEM((CCH, D), jnp.float32),
            pltpu.SemaphoreType.DMA,
        ],
    )
    return scatter, gather, combine


def kernel(x, w_gate, c_fc, c_proj):
    x2 = x.reshape(T, D)
    slot0, slot1, keep0, keep1, w0r, w1r = _route(x2, w_gate)
    scatter, gather, combine = _sc_kernels()
    slot_tok = scatter(slot0, slot1, keep0, keep1)
    exp_x = gather(slot_tok, x2)
    exp_out = _mlp(exp_x, c_fc, c_proj)
    out = combine(slot0, slot1, w0r, w1r, exp_out)
    return out.reshape(1, T, D)
